# baseline (speedup 1.0000x reference)
import functools

import jax
import jax.numpy as jnp
from jax import lax
from jax.experimental import pallas as pl
from jax.experimental.pallas import tpu as pltpu
from jax.experimental.pallas import tpu_sc as plsc

B = 16384
D = 20
NC = 2
NS = 16
L = 16
NW = NC * NS
BPW = B // NW
CHUNK = 128
NCH = BPW // CHUNK
NR = 3
U_OOV = 1000000
UR8 = 2500002
IR8 = 2500000


def _make_kernel():
  mesh = plsc.VectorSubcoreMesh(core_axis_name="c", subcore_axis_name="s")

  @functools.partial(
      pl.kernel,
      mesh=mesh,
      compiler_params=pltpu.CompilerParams(
          needs_layout_passes=False, use_tc_tiling_on_sc=False),
      out_type=jax.ShapeDtypeStruct((B,), jnp.float32),
      scratch_types=[
          pltpu.VMEM((BPW,), jnp.int32),
          pltpu.VMEM((BPW,), jnp.int32),
          pltpu.VMEM((NCH * NR, CHUNK), jnp.int32),
          pltpu.VMEM((NCH * NR, CHUNK), jnp.int32),
          pltpu.VMEM((NCH * NR * CHUNK, 8), jnp.float32),
          pltpu.VMEM((NCH * NR * CHUNK, 8), jnp.float32),
          pltpu.VMEM((32,), jnp.float32),
          pltpu.VMEM((BPW,), jnp.float32),
          pltpu.SemaphoreType.DMA,
      ],
  )
  def k(uidx_hbm, iidx_hbm, ut8_hbm, it8_hbm, oov_hbm, out_hbm,
        uidx_v, iidx_v, uw_v, iw_v, ug_v, ig_v, oov_v, out_v, sem):
    wid = lax.axis_index("s") * NC + lax.axis_index("c")
    base = wid * BPW

    pltpu.sync_copy(uidx_hbm.at[pl.ds(base, BPW)], uidx_v)
    pltpu.sync_copy(iidx_hbm.at[pl.ds(base, BPW)], iidx_v)
    pltpu.sync_copy(oov_hbm, oov_v)

    for c in range(NCH):
      for l in range(CHUNK // L):
        uiv = uidx_v[pl.ds(c * CHUNK + l * L, L)]
        iiv = iidx_v[pl.ds(c * CHUNK + l * L, L)]
        uw = lax.shift_right_logical(uiv * 5, 1)
        iw = lax.shift_right_logical(iiv * 5, 1)
        for r in range(NR):
          uw_v[c * NR + r, pl.ds(l * L, L)] = jnp.minimum(uw + r, UR8 - 1)
          iw_v[c * NR + r, pl.ds(l * L, L)] = iw + r

    cps = []
    for m in range(NCH * NR):
      cps.append(pltpu.async_copy(
          ut8_hbm.at[uw_v.at[m]], ug_v.at[pl.ds(m * CHUNK, CHUNK)], sem))
      cps.append(pltpu.async_copy(
          it8_hbm.at[iw_v.at[m]], ig_v.at[pl.ds(m * CHUNK, CHUNK)], sem))
    for cp in cps:
      cp.wait()

    iot = lax.iota(jnp.int32, L)

    def body(g, carry):
      c = g // (CHUNK // L)
      t = (g % (CHUNK // L)) * L + iot
      uiv = uidx_v[pl.ds(g * L, L)]
      iiv = iidx_v[pl.ds(g * L, L)]
      uoff = (uiv & 1) * 4
      ioff = (iiv & 1) * 4
      ubase = c * (NR * CHUNK) + t
      sel = uiv == U_OOV
      acc = jnp.zeros((L,), jnp.float32)
      for d in range(D):
        ud = uoff + d
        idd = ioff + d
        uu = plsc.load_gather(
            ug_v, [lax.shift_right_logical(ud, 3) * CHUNK + ubase, ud & 7])
        vv = plsc.load_gather(
            ig_v, [lax.shift_right_logical(idd, 3) * CHUNK + ubase, idd & 7])
        ov = plsc.load_gather(oov_v, [jnp.full((L,), d, jnp.int32)])
        acc = acc + jnp.where(sel, ov, uu) * vv
      out_v[pl.ds(g * L, L)] = acc
      return carry

    lax.fori_loop(0, BPW // L, body, 0)

    pltpu.sync_copy(out_v, out_hbm.at[pl.ds(base, BPW)])

  return k


def kernel(user_idx, item_idx, user_table, item_table):
  uflat = jnp.reshape(user_table, (-1,))
  ut8 = jnp.reshape(lax.slice(uflat, (0,), (UR8 * 8,)), (UR8, 8))
  it8 = jnp.reshape(item_table, (IR8, 8))
  oov = jnp.pad(user_table[U_OOV], (0, 32 - D))
  out = _make_kernel()(
      user_idx.astype(jnp.int32), item_idx.astype(jnp.int32),
      ut8, it8, oov)
  return out[:, None]
